# baseline (speedup 1.0000x reference)
import functools

import jax
import jax.numpy as jnp
from jax import lax
from jax.experimental import pallas as pl
from jax.experimental.pallas import tpu as pltpu
from jax.experimental.pallas import tpu_sc as plsc

B, S, H = 2, 2048, 1024
E, K, F = 8, 2, 1024
T = B * S
A = T * K
BS = 256
NB = A // BS + E
NBP = 48
NR = NB * BS
NC, NS, L = 2, 16, 16
NW = NC * NS
TPW = T // NW
RB = 512

def _mesh():
    return plsc.VectorSubcoreMesh(
        core_axis_name="c", subcore_axis_name="s", num_cores=NC, num_subcores=NS
    )


def _bcast_elem(ref, idx):
    return plsc.load_gather(ref, [jnp.zeros((16,), jnp.int32) + idx])


def _top2(pb, n):
    idx = lax.broadcasted_iota(jnp.int32, (n, E), 1)
    m0 = jnp.max(pb, axis=1, keepdims=True)
    i0 = jnp.min(jnp.where(pb == m0, idx, E), axis=1, keepdims=True)
    pb2 = jnp.where(idx == i0, -jnp.inf, pb)
    m1 = jnp.max(pb2, axis=1, keepdims=True)
    i1 = jnp.min(jnp.where(pb2 == m1, idx, E), axis=1, keepdims=True)
    oh0 = (idx == i0).astype(jnp.float32)
    oh1 = (idx == i1).astype(jnp.float32)
    return m0, m1, oh0, oh1


def _route_body(x_ref, wg_ref, pos0_ref, pos1_ref, w0_ref, w1_ref, blk_ref,
                probs_ref, offs_ref, carry_ref, sums_ref, lt_ref):
    phase = pl.program_id(0)
    b = pl.program_id(1)

    @pl.when(phase == 0)
    def _pass1():
        logits = jnp.dot(x_ref[...], wg_ref[...], preferred_element_type=jnp.float32)
        probs = jax.nn.softmax(logits, axis=-1)
        probs_ref[pl.ds(b * RB, RB), :] = probs
        _, _, oh0, oh1 = _top2(probs, RB)
        part = jnp.sum(oh0 + oh1, axis=0, keepdims=True)

        @pl.when(b == 0)
        def _init():
            sums_ref[...] = jnp.zeros_like(sums_ref)
            rr = lax.broadcasted_iota(jnp.int32, (RB, RB), 0)
            cc = lax.broadcasted_iota(jnp.int32, (RB, RB), 1)
            lt_ref[...] = (cc < rr).astype(jnp.float32)

        sums_ref[...] += part

    @pl.when(jnp.logical_and(phase == 1, b == 0))
    def _segments():
        tot = sums_ref[...]
        pc = jnp.floor((tot + (BS - 1)) / BS) * BS
        r = lax.broadcasted_iota(jnp.int32, (E, E), 0)
        c = lax.broadcasted_iota(jnp.int32, (E, E), 1)
        ut_incl = (r <= c).astype(jnp.float32)
        pc8 = jnp.broadcast_to(pc, (E, E))
        ends = jnp.dot(pc8, ut_incl, preferred_element_type=jnp.float32)[0:1, :]
        offs_ref[...] = ends - pc
        carry_ref[...] = jnp.zeros_like(carry_ref)
        bstart = lax.broadcasted_iota(jnp.int32, (NBP, 1), 0).astype(jnp.float32) * BS
        acc = jnp.sum((bstart >= ends).astype(jnp.float32), axis=1, keepdims=True)
        val = jnp.where(bstart < ends[:, E - 1:E], acc, float(E))
        blk_ref[...] = val.astype(jnp.int32)

    @pl.when(phase == 1)
    def _positions():
        m0, m1, oh0, oh1 = _top2(probs_ref[pl.ds(b * RB, RB), :], RB)
        both = oh0 + oh1
        cb = jnp.dot(lt_ref[...], both, preferred_element_type=jnp.float32) + carry_ref[...]
        base0 = offs_ref[...] + cb
        pos0 = jnp.sum(oh0 * base0, axis=1, keepdims=True)
        pos1 = jnp.sum(oh1 * (base0 + oh0), axis=1, keepdims=True)
        pos0_ref[...] = pos0.astype(jnp.int32)
        pos1_ref[...] = pos1.astype(jnp.int32)
        wsum = m0 + m1
        w0_ref[...] = m0 / wsum
        w1_ref[...] = m1 / wsum
        carry_ref[...] += jnp.sum(both, axis=0, keepdims=True)


def _route(x, W_g):
    return pl.pallas_call(
        _route_body,
        grid=(2, T // RB),
        in_specs=[
            pl.BlockSpec((RB, H), lambda p, b: (b * (1 - p), 0)),
            pl.BlockSpec((H, E), lambda p, b: (0, 0)),
        ],
        out_specs=[
            pl.BlockSpec((RB, 1), lambda p, b: (b, 0)),
            pl.BlockSpec((RB, 1), lambda p, b: (b, 0)),
            pl.BlockSpec((RB, 1), lambda p, b: (b, 0)),
            pl.BlockSpec((RB, 1), lambda p, b: (b, 0)),
            pl.BlockSpec((NBP, 1), lambda p, b: (0, 0)),
        ],
        out_shape=[
            jax.ShapeDtypeStruct((T, 1), jnp.int32),
            jax.ShapeDtypeStruct((T, 1), jnp.int32),
            jax.ShapeDtypeStruct((T, 1), jnp.float32),
            jax.ShapeDtypeStruct((T, 1), jnp.float32),
            jax.ShapeDtypeStruct((NBP, 1), jnp.int32),
        ],
        scratch_shapes=[
            pltpu.VMEM((T, E), jnp.float32),
            pltpu.VMEM((1, E), jnp.float32),
            pltpu.VMEM((1, E), jnp.float32),
            pltpu.VMEM((1, E), jnp.float32),
            pltpu.VMEM((RB, RB), jnp.float32),
        ],
    )(x, W_g)


H2 = H // 2


def _dispatch(pos0, pos1, x):
    k = functools.partial(
        pl.kernel,
        out_type=jax.ShapeDtypeStruct((NR, H2), jnp.int32),
        mesh=_mesh(),
        compiler_params=pltpu.CompilerParams(needs_layout_passes=False),
        scratch_types=(
            pltpu.VMEM((TPW,), jnp.int32),
            pltpu.VMEM((TPW,), jnp.int32),
            pltpu.VMEM((4, 32), jnp.int32),
            pltpu.VMEM((4, 32), jnp.int32),
            pltpu.VMEM((4, 32), jnp.int32),
            pltpu.VMEM((32, H2), jnp.int32),
            pltpu.VMEM((32, H2), jnp.int32),
            pltpu.SemaphoreType.DMA,
            pltpu.SemaphoreType.DMA,
        ),
    )(_dispatch_body)
    return k(pos0, pos1, x)


def _dispatch_body(pos0_hbm, pos1_hbm, x_hbm, xs_hbm,
              p0_v, p1_v, p02d_v, p12d_v, tok_v, xba_v, xbb_v, gsem, ssem):
    wid = lax.axis_index("s") * NC + lax.axis_index("c")
    base = wid * TPW
    iota = lax.iota(jnp.int32, 16)

    pltpu.sync_copy(pos0_hbm.at[wid], p0_v)
    pltpu.sync_copy(pos1_hbm.at[wid], p1_v)
    for c in range(4):
        for hh in range(2):
            sl_src = pl.ds(c * 32 + hh * 16, 16)
            sl_dst = pl.ds(hh * 16, 16)
            p02d_v[c, sl_dst] = p0_v[sl_src]
            p12d_v[c, sl_dst] = p1_v[sl_src]
            tok_v[c, sl_dst] = base + c * 32 + hh * 16 + iota

    bufs = (xba_v, xbb_v)
    for c in range(4):
        xb = bufs[c % 2]
        pltpu.async_copy(x_hbm.at[tok_v.at[c]], xb, gsem).wait()
        pltpu.async_copy(xb, xs_hbm.at[p02d_v.at[c]], ssem).wait()
        pltpu.async_copy(xb, xs_hbm.at[p12d_v.at[c]], ssem).wait()


def _mlp_body(blk_ref, xs_ref, w1_ref, b1_ref, w2_ref, b2_ref, ys_ref):
    b = pl.program_id(0)

    @pl.when(blk_ref[b] < E)
    def _run():
        h = jnp.maximum(
            jnp.dot(xs_ref[...], w1_ref[0], preferred_element_type=jnp.float32)
            + b1_ref[0],
            0.0,
        )
        ys_ref[...] = (
            jnp.dot(
                h.astype(jnp.bfloat16),
                w2_ref[0],
                preferred_element_type=jnp.float32,
            )
            + b2_ref[0]
        )


def _mlp(blk, xs, W1, b1, W2, b2):
    grid_spec = pltpu.PrefetchScalarGridSpec(
        num_scalar_prefetch=1,
        grid=(NB,),
        in_specs=[
            pl.BlockSpec((BS, H), lambda b, blk: (b, 0)),
            pl.BlockSpec((1, H, F), lambda b, blk: (jnp.minimum(blk[b], E - 1), 0, 0)),
            pl.BlockSpec((1, 1, F), lambda b, blk: (jnp.minimum(blk[b], E - 1), 0, 0)),
            pl.BlockSpec((1, F, H), lambda b, blk: (jnp.minimum(blk[b], E - 1), 0, 0)),
            pl.BlockSpec((1, 1, H), lambda b, blk: (jnp.minimum(blk[b], E - 1), 0, 0)),
        ],
        out_specs=pl.BlockSpec((BS, H), lambda b, blk: (b, 0)),
    )
    return pl.pallas_call(
        _mlp_body,
        grid_spec=grid_spec,
        out_shape=jax.ShapeDtypeStruct((NR, H), jnp.float32),
    )(blk, xs, W1, b1[:, None, :], W2, b2[:, None, :])


def _combine(ys, pos0, pos1, w0, w1):
    k = functools.partial(
        pl.kernel,
        out_type=jax.ShapeDtypeStruct((T, H), jnp.float32),
        mesh=_mesh(),
        compiler_params=pltpu.CompilerParams(needs_layout_passes=False),
        scratch_types=(
            pltpu.VMEM((TPW,), jnp.int32),
            pltpu.VMEM((TPW,), jnp.int32),
            pltpu.VMEM((TPW,), jnp.float32),
            pltpu.VMEM((TPW,), jnp.float32),
            pltpu.VMEM((4, 32), jnp.int32),
            pltpu.VMEM((4, 32), jnp.int32),
            pltpu.VMEM((32, H), jnp.float32),
            pltpu.VMEM((32, H), jnp.float32),
            pltpu.VMEM((32, H), jnp.float32),
            pltpu.SemaphoreType.DMA,
            pltpu.SemaphoreType.DMA,
        ),
    )(_combine_body)
    return k(ys, pos0, pos1, w0, w1)


def _combine_body(ys_hbm, pos0_hbm, pos1_hbm, w0_hbm, w1_hbm, out_hbm,
             p0_v, p1_v, w0_v, w1_v, p02d_v, p12d_v, ra_v, rb_v, ob_v,
             sem0, sem1):
    wid = lax.axis_index("s") * NC + lax.axis_index("c")
    base = wid * TPW

    pltpu.sync_copy(pos0_hbm.at[wid], p0_v)
    pltpu.sync_copy(pos1_hbm.at[wid], p1_v)
    pltpu.sync_copy(w0_hbm.at[wid], w0_v)
    pltpu.sync_copy(w1_hbm.at[wid], w1_v)
    for c in range(4):
        for hh in range(2):
            p02d_v[c, pl.ds(hh * 16, 16)] = p0_v[pl.ds(c * 32 + hh * 16, 16)]
            p12d_v[c, pl.ds(hh * 16, 16)] = p1_v[pl.ds(c * 32 + hh * 16, 16)]

    for c in range(4):
        cpa = pltpu.async_copy(ys_hbm.at[p02d_v.at[c]], ra_v, sem0)
        cpb = pltpu.async_copy(ys_hbm.at[p12d_v.at[c]], rb_v, sem1)
        cpa.wait()
        cpb.wait()

        def tok_body(i, _):
            wa = _bcast_elem(w0_v, c * 32 + i)
            wb = _bcast_elem(w1_v, c * 32 + i)
            for j in range(H // 16):
                sl = pl.ds(j * 16, 16)
                ob_v[i, sl] = ra_v[i, sl] * wa + rb_v[i, sl] * wb
            return 0

        lax.fori_loop(0, 32, tok_body, 0)
        pltpu.sync_copy(ob_v, out_hbm.at[pl.ds(base + c * 32, 32), :])


@jax.jit
def _moe(x, W_g, W1, b1, W2, b2):
    pos0, pos1, w0, w1, blk = _route(x, W_g)
    pos0 = pos0.reshape(NW, TPW)
    pos1 = pos1.reshape(NW, TPW)
    w0 = w0.reshape(NW, TPW)
    w1 = w1.reshape(NW, TPW)
    blk = blk.reshape(-1)
    x16 = x.astype(jnp.bfloat16)
    xi = lax.bitcast_convert_type(x16.reshape(T, H2, 2), jnp.int32)
    xsi = _dispatch(pos0, pos1, xi)
    xs = lax.bitcast_convert_type(xsi, jnp.bfloat16).reshape(NR, H)
    ys = _mlp(blk, xs, W1.astype(jnp.bfloat16), b1, W2.astype(jnp.bfloat16), b2)
    return _combine(ys, pos0, pos1, w0, w1)


def kernel(hidden_states, W_g, W1, b1, W2, b2):
    orig_shape = hidden_states.shape
    x = hidden_states.reshape(-1, orig_shape[-1])
    out = _moe(x, W_g, W1, b1, W2, b2)
    return out.reshape(orig_shape)


# speedup vs baseline: 2.8402x; 2.8402x over previous
import functools

import jax
import jax.numpy as jnp
from jax import lax
from jax.experimental import pallas as pl
from jax.experimental.pallas import tpu as pltpu
from jax.experimental.pallas import tpu_sc as plsc

B, S, H = 2, 2048, 1024
E, K, F = 8, 2, 1024
T = B * S
A = T * K
BS = 256
NB = A // BS + E
NBP = 48
NR = NB * BS
NC, NS, L = 2, 16, 16
NW = NC * NS
TPW = T // NW
RB = 512

def _mesh():
    return plsc.VectorSubcoreMesh(
        core_axis_name="c", subcore_axis_name="s", num_cores=NC, num_subcores=NS
    )


def _bcast_elem(ref, idx):
    return plsc.load_gather(ref, [jnp.zeros((16,), jnp.int32) + idx])


def _top2(pb, n):
    idx = lax.broadcasted_iota(jnp.int32, (n, E), 1)
    m0 = jnp.max(pb, axis=1, keepdims=True)
    i0 = jnp.min(jnp.where(pb == m0, idx, E), axis=1, keepdims=True)
    pb2 = jnp.where(idx == i0, -jnp.inf, pb)
    m1 = jnp.max(pb2, axis=1, keepdims=True)
    i1 = jnp.min(jnp.where(pb2 == m1, idx, E), axis=1, keepdims=True)
    oh0 = (idx == i0).astype(jnp.float32)
    oh1 = (idx == i1).astype(jnp.float32)
    return m0, m1, oh0, oh1


def _route_body(x_ref, wg_ref, pos0_ref, pos1_ref, w0_ref, w1_ref, blk_ref,
                probs_ref, offs_ref, carry_ref, sums_ref, lt_ref):
    phase = pl.program_id(0)
    b = pl.program_id(1)

    @pl.when(phase == 0)
    def _pass1():
        logits = jnp.dot(x_ref[...], wg_ref[...], preferred_element_type=jnp.float32)
        probs = jax.nn.softmax(logits, axis=-1)
        probs_ref[pl.ds(b * RB, RB), :] = probs
        _, _, oh0, oh1 = _top2(probs, RB)
        part = jnp.sum(oh0 + oh1, axis=0, keepdims=True)

        @pl.when(b == 0)
        def _init():
            sums_ref[...] = jnp.zeros_like(sums_ref)
            rr = lax.broadcasted_iota(jnp.int32, (RB, RB), 0)
            cc = lax.broadcasted_iota(jnp.int32, (RB, RB), 1)
            lt_ref[...] = (cc < rr).astype(jnp.float32)

        sums_ref[...] += part

    @pl.when(jnp.logical_and(phase == 1, b == 0))
    def _segments():
        tot = sums_ref[...]
        pc = jnp.floor((tot + (BS - 1)) / BS) * BS
        r = lax.broadcasted_iota(jnp.int32, (E, E), 0)
        c = lax.broadcasted_iota(jnp.int32, (E, E), 1)
        ut_incl = (r <= c).astype(jnp.float32)
        pc8 = jnp.broadcast_to(pc, (E, E))
        ends = jnp.dot(pc8, ut_incl, preferred_element_type=jnp.float32)[0:1, :]
        offs_ref[...] = ends - pc
        carry_ref[...] = jnp.zeros_like(carry_ref)
        bstart = lax.broadcasted_iota(jnp.int32, (NBP, 1), 0).astype(jnp.float32) * BS
        acc = jnp.sum((bstart >= ends).astype(jnp.float32), axis=1, keepdims=True)
        val = jnp.where(bstart < ends[:, E - 1:E], acc, float(E))
        blk_ref[...] = val.astype(jnp.int32)

    @pl.when(phase == 1)
    def _positions():
        m0, m1, oh0, oh1 = _top2(probs_ref[pl.ds(b * RB, RB), :], RB)
        both = oh0 + oh1
        cb = jnp.dot(lt_ref[...], both, preferred_element_type=jnp.float32) + carry_ref[...]
        base0 = offs_ref[...] + cb
        pos0 = jnp.sum(oh0 * base0, axis=1, keepdims=True)
        pos1 = jnp.sum(oh1 * (base0 + oh0), axis=1, keepdims=True)
        pos0_ref[...] = pos0.astype(jnp.int32)
        pos1_ref[...] = pos1.astype(jnp.int32)
        wsum = m0 + m1
        w0_ref[...] = m0 / wsum
        w1_ref[...] = m1 / wsum
        carry_ref[...] += jnp.sum(both, axis=0, keepdims=True)


def _route(x, W_g):
    return pl.pallas_call(
        _route_body,
        grid=(2, T // RB),
        in_specs=[
            pl.BlockSpec((RB, H), lambda p, b: (b * (1 - p), 0)),
            pl.BlockSpec((H, E), lambda p, b: (0, 0)),
        ],
        out_specs=[
            pl.BlockSpec((RB, 1), lambda p, b: (b, 0)),
            pl.BlockSpec((RB, 1), lambda p, b: (b, 0)),
            pl.BlockSpec((RB, 1), lambda p, b: (b, 0)),
            pl.BlockSpec((RB, 1), lambda p, b: (b, 0)),
            pl.BlockSpec((NBP, 1), lambda p, b: (0, 0)),
        ],
        out_shape=[
            jax.ShapeDtypeStruct((T, 1), jnp.int32),
            jax.ShapeDtypeStruct((T, 1), jnp.int32),
            jax.ShapeDtypeStruct((T, 1), jnp.float32),
            jax.ShapeDtypeStruct((T, 1), jnp.float32),
            jax.ShapeDtypeStruct((NBP, 1), jnp.int32),
        ],
        scratch_shapes=[
            pltpu.VMEM((T, E), jnp.float32),
            pltpu.VMEM((1, E), jnp.float32),
            pltpu.VMEM((1, E), jnp.float32),
            pltpu.VMEM((1, E), jnp.float32),
            pltpu.VMEM((RB, RB), jnp.float32),
        ],
    )(x, W_g)


def _dispatch(pos0, pos1, x):
    k = functools.partial(
        pl.kernel,
        out_type=jax.ShapeDtypeStruct((NR, H), jnp.float32),
        mesh=_mesh(),
        compiler_params=pltpu.CompilerParams(needs_layout_passes=False),
        scratch_types=(
            pltpu.VMEM((TPW,), jnp.int32),
            pltpu.VMEM((TPW,), jnp.int32),
            pltpu.VMEM((4, 32), jnp.int32),
            pltpu.VMEM((4, 32), jnp.int32),
            pltpu.VMEM((4, 32), jnp.int32),
            pltpu.VMEM((32, H), jnp.float32),
            pltpu.VMEM((32, H), jnp.float32),
            pltpu.SemaphoreType.DMA,
            pltpu.SemaphoreType.DMA,
        ),
    )(_dispatch_body)
    return k(pos0, pos1, x)


def _dispatch_body(pos0_hbm, pos1_hbm, x_hbm, xs_hbm,
              p0_v, p1_v, p02d_v, p12d_v, tok_v, xba_v, xbb_v, gsem, ssem):
    wid = lax.axis_index("s") * NC + lax.axis_index("c")
    base = wid * TPW
    iota = lax.iota(jnp.int32, 16)

    pltpu.sync_copy(pos0_hbm.at[wid], p0_v)
    pltpu.sync_copy(pos1_hbm.at[wid], p1_v)
    for c in range(4):
        for hh in range(2):
            sl_src = pl.ds(c * 32 + hh * 16, 16)
            sl_dst = pl.ds(hh * 16, 16)
            p02d_v[c, sl_dst] = p0_v[sl_src]
            p12d_v[c, sl_dst] = p1_v[sl_src]
            tok_v[c, sl_dst] = base + c * 32 + hh * 16 + iota

    bufs = (xba_v, xbb_v)
    for c in range(4):
        xb = bufs[c % 2]
        pltpu.async_copy(x_hbm.at[tok_v.at[c]], xb, gsem).wait()
        pltpu.async_copy(xb, xs_hbm.at[p02d_v.at[c]], ssem).wait()
        pltpu.async_copy(xb, xs_hbm.at[p12d_v.at[c]], ssem).wait()


def _mlp_body(blk_ref, xs_ref, w1_ref, b1_ref, w2_ref, b2_ref, ys_ref):
    b = pl.program_id(0)

    @pl.when(blk_ref[b] < E)
    def _run():
        h = jnp.maximum(
            jnp.dot(
                xs_ref[...].astype(jnp.bfloat16),
                w1_ref[0].astype(jnp.bfloat16),
                preferred_element_type=jnp.float32,
            )
            + b1_ref[0],
            0.0,
        )
        ys_ref[...] = (
            jnp.dot(
                h.astype(jnp.bfloat16),
                w2_ref[0].astype(jnp.bfloat16),
                preferred_element_type=jnp.float32,
            )
            + b2_ref[0]
        )


def _mlp(blk, xs, W1, b1, W2, b2):
    grid_spec = pltpu.PrefetchScalarGridSpec(
        num_scalar_prefetch=1,
        grid=(NB,),
        in_specs=[
            pl.BlockSpec((BS, H), lambda b, blk: (b, 0)),
            pl.BlockSpec((1, H, F), lambda b, blk: (jnp.minimum(blk[b], E - 1), 0, 0)),
            pl.BlockSpec((1, 1, F), lambda b, blk: (jnp.minimum(blk[b], E - 1), 0, 0)),
            pl.BlockSpec((1, F, H), lambda b, blk: (jnp.minimum(blk[b], E - 1), 0, 0)),
            pl.BlockSpec((1, 1, H), lambda b, blk: (jnp.minimum(blk[b], E - 1), 0, 0)),
        ],
        out_specs=pl.BlockSpec((BS, H), lambda b, blk: (b, 0)),
    )
    return pl.pallas_call(
        _mlp_body,
        grid_spec=grid_spec,
        out_shape=jax.ShapeDtypeStruct((NR, H), jnp.float32),
    )(blk, xs, W1, b1[:, None, :], W2, b2[:, None, :])


def _combine(ys, pos0, pos1, w0, w1):
    k = functools.partial(
        pl.kernel,
        out_type=jax.ShapeDtypeStruct((T, H), jnp.float32),
        mesh=_mesh(),
        compiler_params=pltpu.CompilerParams(needs_layout_passes=False),
        scratch_types=(
            pltpu.VMEM((TPW,), jnp.int32),
            pltpu.VMEM((TPW,), jnp.int32),
            pltpu.VMEM((TPW,), jnp.float32),
            pltpu.VMEM((TPW,), jnp.float32),
            pltpu.VMEM((8, 16), jnp.int32),
            pltpu.VMEM((8, 16), jnp.int32),
            pltpu.VMEM((16, H), jnp.float32),
            pltpu.VMEM((16, H), jnp.float32),
            pltpu.VMEM((16, H), jnp.float32),
            pltpu.VMEM((16, H), jnp.float32),
            pltpu.VMEM((16, H), jnp.float32),
            pltpu.SemaphoreType.DMA,
            pltpu.SemaphoreType.DMA,
        ),
    )(_combine_body)
    return k(ys, pos0, pos1, w0, w1)


def _combine_body(ys_hbm, pos0_hbm, pos1_hbm, w0_hbm, w1_hbm, out_hbm,
             p0_v, p1_v, w0_v, w1_v, p02d_v, p12d_v,
             ra0_v, rb0_v, ra1_v, rb1_v, ob_v, sem0, sem1):
    wid = lax.axis_index("s") * NC + lax.axis_index("c")
    base = wid * TPW

    pltpu.sync_copy(pos0_hbm.at[wid], p0_v)
    pltpu.sync_copy(pos1_hbm.at[wid], p1_v)
    pltpu.sync_copy(w0_hbm.at[wid], w0_v)
    pltpu.sync_copy(w1_hbm.at[wid], w1_v)
    for c in range(8):
        p02d_v[c, :] = p0_v[pl.ds(c * 16, 16)]
        p12d_v[c, :] = p1_v[pl.ds(c * 16, 16)]

    bufs = ((ra0_v, rb0_v), (ra1_v, rb1_v))
    nch = TPW // 16
    pend = pltpu.async_copy(ys_hbm.at[p02d_v.at[0]], ra0_v, sem0)
    pendb = pltpu.async_copy(ys_hbm.at[p12d_v.at[0]], rb0_v, sem1)
    for c in range(nch):
        ra_v, rb_v = bufs[c % 2]
        pend.wait()
        pendb.wait()
        if c + 1 < nch:
            nra, nrb = bufs[(c + 1) % 2]
            pend = pltpu.async_copy(ys_hbm.at[p02d_v.at[c + 1]], nra, sem0)
            pendb = pltpu.async_copy(ys_hbm.at[p12d_v.at[c + 1]], nrb, sem1)

        def tok_body(i, _):
            wa = _bcast_elem(w0_v, c * 16 + i)
            wb = _bcast_elem(w1_v, c * 16 + i)
            for j in range(H // 16):
                sl = pl.ds(j * 16, 16)
                ob_v[i, sl] = ra_v[i, sl] * wa + rb_v[i, sl] * wb
            return 0

        lax.fori_loop(0, 16, tok_body, 0)
        pltpu.sync_copy(ob_v, out_hbm.at[pl.ds(base + c * 16, 16), :])


@jax.jit
def _moe(x, W_g, W1, b1, W2, b2):
    pos0, pos1, w0, w1, blk = _route(x, W_g)
    pos0 = pos0.reshape(NW, TPW)
    pos1 = pos1.reshape(NW, TPW)
    w0 = w0.reshape(NW, TPW)
    w1 = w1.reshape(NW, TPW)
    blk = blk.reshape(-1)
    xs = _dispatch(pos0, pos1, x)
    ys = _mlp(blk, xs, W1, b1, W2, b2)
    return _combine(ys, pos0, pos1, w0, w1)


def kernel(hidden_states, W_g, W1, b1, W2, b2):
    orig_shape = hidden_states.shape
    x = hidden_states.reshape(-1, orig_shape[-1])
    out = _moe(x, W_g, W1, b1, W2, b2)
    return out.reshape(orig_shape)


# speedup vs baseline: 2.8774x; 1.0131x over previous
import functools

import jax
import jax.numpy as jnp
from jax import lax
from jax.experimental import pallas as pl
from jax.experimental.pallas import tpu as pltpu
from jax.experimental.pallas import tpu_sc as plsc

B, S, H = 2, 2048, 1024
E, K, F = 8, 2, 1024
T = B * S
A = T * K
BS = 256
NB = A // BS + E
NBP = 48
NR = NB * BS
NC, NS, L = 2, 16, 16
NW = NC * NS
TPW = T // NW
RB = 512

def _mesh():
    return plsc.VectorSubcoreMesh(
        core_axis_name="c", subcore_axis_name="s", num_cores=NC, num_subcores=NS
    )


def _bcast_elem(ref, idx):
    return plsc.load_gather(ref, [jnp.zeros((16,), jnp.int32) + idx])


def _top2(pb, n):
    idx = lax.broadcasted_iota(jnp.int32, (n, E), 1)
    m0 = jnp.max(pb, axis=1, keepdims=True)
    i0 = jnp.min(jnp.where(pb == m0, idx, E), axis=1, keepdims=True)
    pb2 = jnp.where(idx == i0, -jnp.inf, pb)
    m1 = jnp.max(pb2, axis=1, keepdims=True)
    i1 = jnp.min(jnp.where(pb2 == m1, idx, E), axis=1, keepdims=True)
    oh0 = (idx == i0).astype(jnp.float32)
    oh1 = (idx == i1).astype(jnp.float32)
    return m0, m1, oh0, oh1


def _route_body(x_ref, wg_ref, pos0_ref, pos1_ref, w0_ref, w1_ref, blk_ref,
                probs_ref, offs_ref, carry_ref, sums_ref, lt_ref):
    phase = pl.program_id(0)
    b = pl.program_id(1)

    @pl.when(phase == 0)
    def _pass1():
        logits = jnp.dot(x_ref[...], wg_ref[...], preferred_element_type=jnp.float32)
        probs = jax.nn.softmax(logits, axis=-1)
        probs_ref[pl.ds(b * RB, RB), :] = probs
        _, _, oh0, oh1 = _top2(probs, RB)
        part = jnp.sum(oh0 + oh1, axis=0, keepdims=True)

        @pl.when(b == 0)
        def _init():
            sums_ref[...] = jnp.zeros_like(sums_ref)
            rr = lax.broadcasted_iota(jnp.int32, (RB, RB), 0)
            cc = lax.broadcasted_iota(jnp.int32, (RB, RB), 1)
            lt_ref[...] = (cc < rr).astype(jnp.float32)

        sums_ref[...] += part

    @pl.when(jnp.logical_and(phase == 1, b == 0))
    def _segments():
        tot = sums_ref[...]
        pc = jnp.floor((tot + (BS - 1)) / BS) * BS
        r = lax.broadcasted_iota(jnp.int32, (E, E), 0)
        c = lax.broadcasted_iota(jnp.int32, (E, E), 1)
        ut_incl = (r <= c).astype(jnp.float32)
        pc8 = jnp.broadcast_to(pc, (E, E))
        ends = jnp.dot(pc8, ut_incl, preferred_element_type=jnp.float32)[0:1, :]
        offs_ref[...] = ends - pc
        carry_ref[...] = jnp.zeros_like(carry_ref)
        bstart = lax.broadcasted_iota(jnp.int32, (NBP, 1), 0).astype(jnp.float32) * BS
        acc = jnp.sum((bstart >= ends).astype(jnp.float32), axis=1, keepdims=True)
        val = jnp.where(bstart < ends[:, E - 1:E], acc, float(E))
        blk_ref[...] = val.astype(jnp.int32)

    @pl.when(phase == 1)
    def _positions():
        m0, m1, oh0, oh1 = _top2(probs_ref[pl.ds(b * RB, RB), :], RB)
        both = oh0 + oh1
        cb = jnp.dot(lt_ref[...], both, preferred_element_type=jnp.float32) + carry_ref[...]
        base0 = offs_ref[...] + cb
        pos0 = jnp.sum(oh0 * base0, axis=1, keepdims=True)
        pos1 = jnp.sum(oh1 * (base0 + oh0), axis=1, keepdims=True)
        pos0_ref[...] = pos0.astype(jnp.int32)
        pos1_ref[...] = pos1.astype(jnp.int32)
        wsum = m0 + m1
        w0_ref[...] = m0 / wsum
        w1_ref[...] = m1 / wsum
        carry_ref[...] += jnp.sum(both, axis=0, keepdims=True)


def _route(x, W_g):
    return pl.pallas_call(
        _route_body,
        grid=(2, T // RB),
        in_specs=[
            pl.BlockSpec((RB, H), lambda p, b: (b * (1 - p), 0)),
            pl.BlockSpec((H, E), lambda p, b: (0, 0)),
        ],
        out_specs=[
            pl.BlockSpec((RB, 1), lambda p, b: (b, 0)),
            pl.BlockSpec((RB, 1), lambda p, b: (b, 0)),
            pl.BlockSpec((RB, 1), lambda p, b: (b, 0)),
            pl.BlockSpec((RB, 1), lambda p, b: (b, 0)),
            pl.BlockSpec((NBP, 1), lambda p, b: (0, 0)),
        ],
        out_shape=[
            jax.ShapeDtypeStruct((T, 1), jnp.int32),
            jax.ShapeDtypeStruct((T, 1), jnp.int32),
            jax.ShapeDtypeStruct((T, 1), jnp.float32),
            jax.ShapeDtypeStruct((T, 1), jnp.float32),
            jax.ShapeDtypeStruct((NBP, 1), jnp.int32),
        ],
        scratch_shapes=[
            pltpu.VMEM((T, E), jnp.float32),
            pltpu.VMEM((1, E), jnp.float32),
            pltpu.VMEM((1, E), jnp.float32),
            pltpu.VMEM((1, E), jnp.float32),
            pltpu.VMEM((RB, RB), jnp.float32),
        ],
    )(x, W_g)


def _dispatch(pos0, pos1, x):
    k = functools.partial(
        pl.kernel,
        out_type=jax.ShapeDtypeStruct((NR, H), jnp.float32),
        mesh=_mesh(),
        compiler_params=pltpu.CompilerParams(needs_layout_passes=False),
        scratch_types=(
            pltpu.VMEM((TPW,), jnp.int32),
            pltpu.VMEM((TPW,), jnp.int32),
            pltpu.VMEM((4, 32), jnp.int32),
            pltpu.VMEM((4, 32), jnp.int32),
            pltpu.VMEM((4, 32), jnp.int32),
            pltpu.VMEM((32, H), jnp.float32),
            pltpu.VMEM((32, H), jnp.float32),
            pltpu.SemaphoreType.DMA,
            pltpu.SemaphoreType.DMA,
        ),
    )(_dispatch_body)
    return k(pos0, pos1, x)


def _dispatch_body(pos0_hbm, pos1_hbm, x_hbm, xs_hbm,
              p0_v, p1_v, p02d_v, p12d_v, tok_v, xba_v, xbb_v, gsem, ssem):
    wid = lax.axis_index("s") * NC + lax.axis_index("c")
    base = wid * TPW
    iota = lax.iota(jnp.int32, 16)

    pltpu.sync_copy(pos0_hbm.at[wid], p0_v)
    pltpu.sync_copy(pos1_hbm.at[wid], p1_v)
    for c in range(4):
        for hh in range(2):
            sl_src = pl.ds(c * 32 + hh * 16, 16)
            sl_dst = pl.ds(hh * 16, 16)
            p02d_v[c, sl_dst] = p0_v[sl_src]
            p12d_v[c, sl_dst] = p1_v[sl_src]
            tok_v[c, sl_dst] = base + c * 32 + hh * 16 + iota

    bufs = (xba_v, xbb_v)
    scat = [None] * 4
    pend = pltpu.async_copy(x_hbm.at[tok_v.at[0]], xba_v, gsem)
    for c in range(4):
        xb = bufs[c % 2]
        pend.wait()
        if c + 1 < 4:
            if c - 1 >= 0:
                scat[c - 1][0].wait()
                scat[c - 1][1].wait()
            pend = pltpu.async_copy(x_hbm.at[tok_v.at[c + 1]], bufs[(c + 1) % 2], gsem)
        scat[c] = (
            pltpu.async_copy(xb, xs_hbm.at[p02d_v.at[c]], ssem),
            pltpu.async_copy(xb, xs_hbm.at[p12d_v.at[c]], ssem),
        )
    for c in (2, 3):
        scat[c][0].wait()
        scat[c][1].wait()


def _mlp_body(blk_ref, xs_ref, w1_ref, b1_ref, w2_ref, b2_ref, ys_ref):
    b = pl.program_id(0)

    @pl.when(blk_ref[b] < E)
    def _run():
        h = jnp.maximum(
            jnp.dot(
                xs_ref[...].astype(jnp.bfloat16),
                w1_ref[0].astype(jnp.bfloat16),
                preferred_element_type=jnp.float32,
            )
            + b1_ref[0],
            0.0,
        )
        ys_ref[...] = (
            jnp.dot(
                h.astype(jnp.bfloat16),
                w2_ref[0].astype(jnp.bfloat16),
                preferred_element_type=jnp.float32,
            )
            + b2_ref[0]
        )


def _mlp(blk, xs, W1, b1, W2, b2):
    grid_spec = pltpu.PrefetchScalarGridSpec(
        num_scalar_prefetch=1,
        grid=(NB,),
        in_specs=[
            pl.BlockSpec((BS, H), lambda b, blk: (b, 0)),
            pl.BlockSpec((1, H, F), lambda b, blk: (jnp.minimum(blk[b], E - 1), 0, 0)),
            pl.BlockSpec((1, 1, F), lambda b, blk: (jnp.minimum(blk[b], E - 1), 0, 0)),
            pl.BlockSpec((1, F, H), lambda b, blk: (jnp.minimum(blk[b], E - 1), 0, 0)),
            pl.BlockSpec((1, 1, H), lambda b, blk: (jnp.minimum(blk[b], E - 1), 0, 0)),
        ],
        out_specs=pl.BlockSpec((BS, H), lambda b, blk: (b, 0)),
    )
    return pl.pallas_call(
        _mlp_body,
        grid_spec=grid_spec,
        out_shape=jax.ShapeDtypeStruct((NR, H), jnp.float32),
    )(blk, xs, W1, b1[:, None, :], W2, b2[:, None, :])


def _combine(ys, pos0, pos1, w0, w1):
    k = functools.partial(
        pl.kernel,
        out_type=jax.ShapeDtypeStruct((T, H), jnp.float32),
        mesh=_mesh(),
        compiler_params=pltpu.CompilerParams(needs_layout_passes=False),
        scratch_types=(
            pltpu.VMEM((TPW,), jnp.int32),
            pltpu.VMEM((TPW,), jnp.int32),
            pltpu.VMEM((TPW,), jnp.float32),
            pltpu.VMEM((TPW,), jnp.float32),
            pltpu.VMEM((8, 16), jnp.int32),
            pltpu.VMEM((8, 16), jnp.int32),
            pltpu.VMEM((16, H), jnp.float32),
            pltpu.VMEM((16, H), jnp.float32),
            pltpu.VMEM((16, H), jnp.float32),
            pltpu.VMEM((16, H), jnp.float32),
            pltpu.VMEM((16, H), jnp.float32),
            pltpu.SemaphoreType.DMA,
            pltpu.SemaphoreType.DMA,
        ),
    )(_combine_body)
    return k(ys, pos0, pos1, w0, w1)


def _combine_body(ys_hbm, pos0_hbm, pos1_hbm, w0_hbm, w1_hbm, out_hbm,
             p0_v, p1_v, w0_v, w1_v, p02d_v, p12d_v,
             ra0_v, rb0_v, ra1_v, rb1_v, ob_v, sem0, sem1):
    wid = lax.axis_index("s") * NC + lax.axis_index("c")
    base = wid * TPW

    pltpu.sync_copy(pos0_hbm.at[wid], p0_v)
    pltpu.sync_copy(pos1_hbm.at[wid], p1_v)
    pltpu.sync_copy(w0_hbm.at[wid], w0_v)
    pltpu.sync_copy(w1_hbm.at[wid], w1_v)
    for c in range(8):
        p02d_v[c, :] = p0_v[pl.ds(c * 16, 16)]
        p12d_v[c, :] = p1_v[pl.ds(c * 16, 16)]

    bufs = ((ra0_v, rb0_v), (ra1_v, rb1_v))
    nch = TPW // 16
    pend = pltpu.async_copy(ys_hbm.at[p02d_v.at[0]], ra0_v, sem0)
    pendb = pltpu.async_copy(ys_hbm.at[p12d_v.at[0]], rb0_v, sem1)
    for c in range(nch):
        ra_v, rb_v = bufs[c % 2]
        pend.wait()
        pendb.wait()
        if c + 1 < nch:
            nra, nrb = bufs[(c + 1) % 2]
            pend = pltpu.async_copy(ys_hbm.at[p02d_v.at[c + 1]], nra, sem0)
            pendb = pltpu.async_copy(ys_hbm.at[p12d_v.at[c + 1]], nrb, sem1)

        def tok_body(i, _):
            wa = _bcast_elem(w0_v, c * 16 + i)
            wb = _bcast_elem(w1_v, c * 16 + i)
            for j in range(H // 16):
                sl = pl.ds(j * 16, 16)
                ob_v[i, sl] = ra_v[i, sl] * wa + rb_v[i, sl] * wb
            return 0

        lax.fori_loop(0, 16, tok_body, 0)
        pltpu.sync_copy(ob_v, out_hbm.at[pl.ds(base + c * 16, 16), :])


@jax.jit
def _moe(x, W_g, W1, b1, W2, b2):
    pos0, pos1, w0, w1, blk = _route(x, W_g)
    pos0 = pos0.reshape(NW, TPW)
    pos1 = pos1.reshape(NW, TPW)
    w0 = w0.reshape(NW, TPW)
    w1 = w1.reshape(NW, TPW)
    blk = blk.reshape(-1)
    xs = _dispatch(pos0, pos1, x)
    ys = _mlp(blk, xs, W1, b1, W2, b2)
    return _combine(ys, pos0, pos1, w0, w1)


def kernel(hidden_states, W_g, W1, b1, W2, b2):
    orig_shape = hidden_states.shape
    x = hidden_states.reshape(-1, orig_shape[-1])
    out = _moe(x, W_g, W1, b1, W2, b2)
    return out.reshape(orig_shape)
